# Initial kernel scaffold; baseline (speedup 1.0000x reference)
import jax
import jax.numpy as jnp
from jax.experimental import pallas as pl


def kernel(nodes, metapath_instances, metapath_embedding, features, W1, attn2):
    raise NotImplementedError("write your pallas kernel here")


# speedup vs baseline: 4.7640x; 4.7640x over previous
import functools

import jax
import jax.numpy as jnp
from jax import lax
from jax.experimental import pallas as pl
from jax.experimental.pallas import tpu as pltpu
from jax.experimental.pallas import tpu_sc as plsc

_N = 10000
_E = 320000
_H = 4
_D = 128
_ALPHA = 0.2

_NPW = 320
_NPAD = _NPW * 32
_CH = 64
_RPLEN = (_NPAD - _NPW) + _NPW + 16
_HP = 8


def _attn_matmul_tc(x, w):
    m = x.shape[0]
    be = 2560
    assert m % be == 0

    def body(w_ref, x_ref, o_ref):
        o_ref[...] = lax.dot_general(
            w_ref[...], x_ref[...], (((1,), (1,)), ((), ())),
            preferred_element_type=jnp.float32)

    return pl.pallas_call(
        body,
        grid=(m // be,),
        in_specs=[
            pl.BlockSpec((_HP, _D), lambda i: (0, 0)),
            pl.BlockSpec((be, _D), lambda i: (i, 0)),
        ],
        out_specs=pl.BlockSpec((_HP, be), lambda i: (0, i)),
        out_shape=jax.ShapeDtypeStruct((_HP, m), jnp.float32),
    )(w, x)


def _sc_aggregate(emb, a2t, a1t, rp):
    mesh = plsc.VectorSubcoreMesh(core_axis_name="c", subcore_axis_name="s")
    nt = _D // 16
    npw16 = _NPW + 16
    chp = _CH + 16

    @functools.partial(
        pl.kernel,
        mesh=mesh,
        out_type=jax.ShapeDtypeStruct((_NPAD * _H * _D,), jnp.float32),
        scratch_types=[
            pltpu.VMEM((npw16,), jnp.int32),
            pltpu.VMEM((_H * npw16,), jnp.float32),
            pltpu.VMEM((_CH * _D,), jnp.float32),
            pltpu.VMEM((_H * _CH,), jnp.float32),
            pltpu.VMEM((_H * chp,), jnp.float32),
            pltpu.VMEM((_H * _D,), jnp.float32),
            pltpu.VMEM((_H * _D,), jnp.float32),
            pltpu.SemaphoreType.DMA,
        ],
    )
    def sck(emb_h, a2_h, a1_h, rp_h, out_h,
            rp_v, a1_v, emb_v, a2_v, w_v, s_v, row_v, sem):
        wid = lax.axis_index("s") * 2 + lax.axis_index("c")
        n0 = pl.multiple_of(wid * _NPW, 8)
        pltpu.sync_copy(rp_h.at[pl.ds(n0, npw16)], rp_v)
        for h in range(_H):
            pltpu.sync_copy(a1_h.at[pl.ds(pl.multiple_of(h * _NPAD + n0, 8), _NPW)],
                            a1_v.at[pl.ds(h * npw16, _NPW)])
        iot = lax.iota(jnp.int32, 16)
        neg = jnp.float32(-1e30)
        zv = jnp.zeros((16,), jnp.float32)
        perms = [iot ^ s for s in (1, 2, 4, 8)]

        gdn = lax.GatherDimensionNumbers(
            offset_dims=(), collapsed_slice_dims=(0,), start_index_map=(0,))

        def lperm(x, p):
            return lax.gather(
                x, p[:, None], gdn, (1,),
                mode=lax.GatherScatterMode.PROMISE_IN_BOUNDS)

        def allmax(x):
            for p in perms:
                x = jnp.maximum(x, lperm(x, p))
            return x

        def allsum(x):
            for p in perms:
                x = x + lperm(x, p)
            return x

        def node_body(i, tok):
            rpv = rp_v[pl.ds(i, 16)]
            e_s = rpv[0]
            e_e = rpv[1]
            a1b = [jnp.full((16,), a1_v[pl.ds(h * npw16 + i, 16)][0])
                   for h in range(_H)]
            for h in range(_H):
                for t in range(nt):
                    s_v[pl.ds(h * _D + t * 16, 16)] = zv
            mneg = jnp.full((16,), neg)
            carry0 = (e_s, mneg, mneg, mneg, mneg, zv, zv, zv, zv)
            cs0 = jnp.minimum(e_s & jnp.int32(-8), jnp.int32(_E - _CH))
            nch = jnp.where(e_e > e_s,
                            (e_e - cs0 + jnp.int32(_CH - 1)) // _CH,
                            jnp.int32(0))

            def chunk(_, c):
                s_pos = c[0]
                m = list(c[1:5])
                d = list(c[5:9])
                cs = pl.multiple_of(
                    jnp.minimum(s_pos & jnp.int32(-8), jnp.int32(_E - _CH)), 8)
                cp = [pltpu.async_copy(
                    emb_h.at[pl.ds(pl.multiple_of(cs * _D, 8), _CH * _D)],
                    emb_v, sem)]
                for hh in range(_H):
                    cp.append(pltpu.async_copy(
                        a2_h.at[pl.ds(pl.multiple_of(hh * _E + cs, 8), _CH)],
                        a2_v.at[pl.ds(hh * _CH, _CH)], sem))
                for c2 in cp:
                    c2.wait()
                mc = [jnp.full((16,), neg) for _ in range(_H)]
                for h in range(_H):
                    for kk in range(_CH // 16):
                        eidx = cs + kk * 16 + iot
                        valid = (eidx >= s_pos) & (eidx < e_e)
                        a = a2_v[pl.ds(h * _CH + kk * 16, 16)] + a1b[h]
                        a = jnp.where(a >= 0, a, _ALPHA * a)
                        a = jnp.where(valid, a, neg)
                        w_v[pl.ds(h * chp + kk * 16, 16)] = a
                        mc[h] = jnp.maximum(mc[h], a)
                newm = []
                f = []
                nd = []
                for h in range(_H):
                    nm = jnp.maximum(m[h], allmax(mc[h]))
                    newm.append(nm)
                    f.append(jnp.exp(m[h] - nm))
                for h in range(_H):
                    acc = zv
                    for kk in range(_CH // 16):
                        sl = pl.ds(h * chp + kk * 16, 16)
                        w = jnp.exp(w_v[sl] - newm[h])
                        w_v[sl] = w
                        acc = acc + w
                    nd.append(d[h] * f[h] + allsum(acc))
                jlo = s_pos - cs
                jhi = jnp.minimum(jnp.int32(_CH), e_e - cs)

                def edge_body(j, accs):
                    accs = list(accs)
                    wj = [jnp.full((16,), w_v[pl.ds(h * chp + j, 16)][0])
                          for h in range(_H)]
                    for t in range(nt):
                        em = emb_v[pl.ds(j * _D + t * 16, 16)]
                        for h in range(_H):
                            accs[h * nt + t] = accs[h * nt + t] + wj[h] * em
                    return tuple(accs)

                zacc = tuple(zv for _ in range(_H * nt))
                accs = lax.fori_loop(jlo, jhi, edge_body, zacc)
                for h in range(_H):
                    for t in range(nt):
                        sl = pl.ds(h * _D + t * 16, 16)
                        s_v[sl] = s_v[sl] * f[h] + accs[h * nt + t]
                return (cs + jnp.int32(_CH),
                        newm[0], newm[1], newm[2], newm[3],
                        nd[0], nd[1], nd[2], nd[3])

            fc = lax.fori_loop(0, nch, chunk, carry0)
            for h in range(_H):
                dh = fc[5 + h]
                safe = dh > 0
                invd = 1.0 / jnp.where(safe, dh, 1.0)
                for t in range(nt):
                    r = s_v[pl.ds(h * _D + t * 16, 16)] * invd
                    r = jnp.where(safe, r, 0.0)
                    o = jnp.where(r > 0, r, jnp.exp(r) - 1.0)
                    row_v[pl.ds(h * _D + t * 16, 16)] = o
            pltpu.sync_copy(
                row_v,
                out_h.at[pl.ds(pl.multiple_of((n0 + i) * _H * _D, 8), _H * _D)])
            return tok

        lax.fori_loop(0, _NPW, node_body, 0)

    return sck(emb.reshape(_E * _D), a2t.reshape(_HP * _E)[:_H * _E],
               a1t.reshape(_HP * _NPAD)[:_H * _NPAD], rp)


def kernel(nodes, metapath_instances, metapath_embedding, features, W1, attn2):
    dst = metapath_instances
    emb = metapath_embedding
    w1p = jnp.pad(W1, ((0, _HP - _H), (0, 0)))
    attn2p = jnp.pad(attn2, ((0, _HP - _H), (0, 0)))
    featp = jnp.pad(features, ((0, _NPAD - _N), (0, 0)))
    a2t = _attn_matmul_tc(emb, attn2p)
    a1t = _attn_matmul_tc(featp, w1p)
    rp = jnp.searchsorted(
        dst, jnp.arange(_RPLEN, dtype=jnp.int32), side="left"
    ).astype(jnp.int32)
    out = _sc_aggregate(emb, a2t, a1t, rp)
    return out.reshape(_NPAD, _H * _D)[:_N]


# speedup vs baseline: 8.3322x; 1.7490x over previous
import functools

import jax
import jax.numpy as jnp
from jax import lax
from jax.experimental import pallas as pl
from jax.experimental.pallas import tpu as pltpu
from jax.experimental.pallas import tpu_sc as plsc

_N = 10000
_E = 320000
_H = 4
_D = 128
_ALPHA = 0.2

_NPW = 320
_NPAD = _NPW * 32
_CH = 64
_RPLEN = (_NPAD - _NPW) + _NPW + 16
_HP = 8


def _attn_matmul_tc(x, w):
    m = x.shape[0]
    be = 2560
    assert m % be == 0

    def body(w_ref, x_ref, o_ref):
        o_ref[...] = lax.dot_general(
            w_ref[...], x_ref[...], (((1,), (1,)), ((), ())),
            preferred_element_type=jnp.float32)

    return pl.pallas_call(
        body,
        grid=(m // be,),
        in_specs=[
            pl.BlockSpec((_HP, _D), lambda i: (0, 0)),
            pl.BlockSpec((be, _D), lambda i: (i, 0)),
        ],
        out_specs=pl.BlockSpec((_HP, be), lambda i: (0, i)),
        out_shape=jax.ShapeDtypeStruct((_HP, m), jnp.float32),
    )(w, x)


def _sc_aggregate(emb, a2t, a1t, dst):
    mesh = plsc.VectorSubcoreMesh(core_axis_name="c", subcore_axis_name="s")
    nt = _D // 16
    npw16 = _NPW + 16
    chp = _CH + 16
    chd = 256

    @functools.partial(
        pl.kernel,
        mesh=mesh,
        compiler_params=pltpu.CompilerParams(needs_layout_passes=False),
        out_type=jax.ShapeDtypeStruct((_NPAD * _H * _D,), jnp.float32),
        scratch_types=[
            pltpu.VMEM((npw16,), jnp.int32),
            pltpu.VMEM((_H * npw16,), jnp.float32),
            pltpu.VMEM((_CH * _D,), jnp.float32),
            pltpu.VMEM((_H * _CH,), jnp.float32),
            pltpu.VMEM((_H * chp,), jnp.float32),
            pltpu.VMEM((_H * _D,), jnp.float32),
            pltpu.VMEM((_H * _D,), jnp.float32),
            pltpu.VMEM((16,), jnp.int32),
            pltpu.VMEM((16,), jnp.int32),
            pltpu.VMEM((chd,), jnp.int32),
            pltpu.SemaphoreType.DMA,
        ],
    )
    def sck(emb_h, a2_h, a1_h, dst_h, out_h,
            rp_v, a1_v, emb_v, a2_v, w_v, s_v, row_v, p1_v, p2_v, dc_v, sem):
        wid = lax.axis_index("s") * 2 + lax.axis_index("c")
        n0 = pl.multiple_of(wid * _NPW, 8)
        for h in range(_H):
            pltpu.sync_copy(a1_h.at[pl.ds(pl.multiple_of(h * _NPAD + n0, 8), _NPW)],
                            a1_v.at[pl.ds(h * npw16, _NPW)])
        iot = lax.iota(jnp.int32, 16)
        neg = jnp.float32(-1e30)
        zv = jnp.zeros((16,), jnp.float32)
        perms = [iot ^ s for s in (1, 2, 4, 8)]

        gdn = lax.GatherDimensionNumbers(
            offset_dims=(), collapsed_slice_dims=(0,), start_index_map=(0,))

        def lperm(x, p):
            return lax.gather(
                x, p[:, None], gdn, (1,),
                mode=lax.GatherScatterMode.PROMISE_IN_BOUNDS)

        def allmax(x):
            for p in perms:
                x = jnp.maximum(x, lperm(x, p))
            return x

        def allsum(x):
            for p in perms:
                x = x + lperm(x, p)
            return x

        v1 = n0
        v2 = n0 + jnp.int32(_NPW)
        lane15 = jnp.full((16,), 15, jnp.int32)

        def bs_body(_, c):
            lo1, hi1, lo2, hi2 = c
            mid1 = (lo1 + hi1) // 2
            mid2 = (lo2 + hi2) // 2
            b1 = pl.multiple_of(
                jnp.minimum(mid1 & jnp.int32(-8), jnp.int32(_E - 16)), 8)
            b2 = pl.multiple_of(
                jnp.minimum(mid2 & jnp.int32(-8), jnp.int32(_E - 16)), 8)
            c1 = pltpu.async_copy(dst_h.at[pl.ds(b1, 16)], p1_v, sem)
            c2 = pltpu.async_copy(dst_h.at[pl.ds(b2, 16)], p2_v, sem)
            c1.wait()
            c2.wait()
            val1 = lperm(p1_v[...], jnp.full((16,), mid1 - b1))[0]
            val2 = lperm(p2_v[...], jnp.full((16,), mid2 - b2))[0]
            pr1 = val1 < v1
            pr2 = val2 < v2
            return (jnp.where(pr1, mid1 + 1, lo1), jnp.where(pr1, hi1, mid1),
                    jnp.where(pr2, mid2 + 1, lo2), jnp.where(pr2, hi2, mid2))

        ze = jnp.int32(0)
        e0w, _, eendw, _ = lax.fori_loop(
            0, 19, bs_body, (ze, jnp.int32(_E), ze, jnp.int32(_E)))

        inf = jnp.full((16,), jnp.int32(2147483647))
        for g in range(npw16 // 16):
            rp_v[pl.ds(g * 16, 16)] = inf
        idxm1 = (iot + 15) & 15
        ndch = jnp.where(
            eendw > e0w,
            (eendw - jnp.minimum(e0w & jnp.int32(-8), jnp.int32(_E - chd))
             + jnp.int32(chd - 1)) // chd,
            ze)

        def rp_chunk(_, c):
            pos, prev = c
            cb = pl.multiple_of(
                jnp.minimum(pos & jnp.int32(-8), jnp.int32(_E - chd)), 8)
            pltpu.sync_copy(dst_h.at[pl.ds(cb, chd)], dc_v)
            for g in range(chd // 16):
                v = dc_v[pl.ds(g * 16, 16)]
                shifted = jnp.where(iot == 0, prev, lperm(v, idxm1))
                eidx = cb + g * 16 + iot
                fm = (v != shifted) & (eidx >= pos) & (eidx < eendw)
                sidx = jnp.where(fm, v - n0, jnp.int32(npw16 - 8))
                sval = jnp.where(fm, eidx, jnp.int32(2147483647))
                plsc.store_scatter(rp_v, [sidx], sval)
                prev = lperm(v, lane15)
            return (cb + jnp.int32(chd), prev)

        lax.fori_loop(0, ndch, rp_chunk,
                      (e0w, jnp.full((16,), jnp.int32(-1))))

        carry_f = jnp.full((16,), eendw)
        for k in range(npw16 // 16):
            g = npw16 // 16 - 1 - k
            sl = pl.ds(g * 16, 16)
            y = lax.rev(rp_v[sl], (0,))
            z = -plsc.cummax(-y)
            z = jnp.minimum(z, carry_f)
            carry_f = lperm(z, lane15)
            rp_v[sl] = lax.rev(z, (0,))

        def node_body(i, tok):
            rpv = rp_v[pl.ds(i, 16)]
            e_s = rpv[0]
            e_e = rpv[1]
            a1b = [jnp.full((16,), a1_v[pl.ds(h * npw16 + i, 16)][0])
                   for h in range(_H)]
            for h in range(_H):
                for t in range(nt):
                    s_v[pl.ds(h * _D + t * 16, 16)] = zv
            mneg = jnp.full((16,), neg)
            carry0 = (e_s, mneg, mneg, mneg, mneg, zv, zv, zv, zv)
            cs0 = jnp.minimum(e_s & jnp.int32(-8), jnp.int32(_E - _CH))
            nch = jnp.where(e_e > e_s,
                            (e_e - cs0 + jnp.int32(_CH - 1)) // _CH,
                            jnp.int32(0))

            def chunk(_, c):
                s_pos = c[0]
                m = list(c[1:5])
                d = list(c[5:9])
                cs = pl.multiple_of(
                    jnp.minimum(s_pos & jnp.int32(-8), jnp.int32(_E - _CH)), 8)
                cp = [pltpu.async_copy(
                    emb_h.at[pl.ds(pl.multiple_of(cs * _D, 8), _CH * _D)],
                    emb_v, sem)]
                for hh in range(_H):
                    cp.append(pltpu.async_copy(
                        a2_h.at[pl.ds(pl.multiple_of(hh * _E + cs, 8), _CH)],
                        a2_v.at[pl.ds(hh * _CH, _CH)], sem))
                for c2 in cp:
                    c2.wait()
                mc = [jnp.full((16,), neg) for _ in range(_H)]
                for h in range(_H):
                    for kk in range(_CH // 16):
                        eidx = cs + kk * 16 + iot
                        valid = (eidx >= s_pos) & (eidx < e_e)
                        a = a2_v[pl.ds(h * _CH + kk * 16, 16)] + a1b[h]
                        a = jnp.where(a >= 0, a, _ALPHA * a)
                        a = jnp.where(valid, a, neg)
                        w_v[pl.ds(h * chp + kk * 16, 16)] = a
                        mc[h] = jnp.maximum(mc[h], a)
                newm = []
                f = []
                nd = []
                for h in range(_H):
                    nm = jnp.maximum(m[h], allmax(mc[h]))
                    newm.append(nm)
                    f.append(jnp.exp(m[h] - nm))
                for h in range(_H):
                    acc = zv
                    for kk in range(_CH // 16):
                        sl = pl.ds(h * chp + kk * 16, 16)
                        w = jnp.exp(w_v[sl] - newm[h])
                        w_v[sl] = w
                        acc = acc + w
                    nd.append(d[h] * f[h] + allsum(acc))
                jlo = s_pos - cs
                jhi = jnp.minimum(jnp.int32(_CH), e_e - cs)

                def edge_body(j, accs):
                    accs = list(accs)
                    wj = [jnp.full((16,), w_v[pl.ds(h * chp + j, 16)][0])
                          for h in range(_H)]
                    for t in range(nt):
                        em = emb_v[pl.ds(j * _D + t * 16, 16)]
                        for h in range(_H):
                            accs[h * nt + t] = accs[h * nt + t] + wj[h] * em
                    return tuple(accs)

                zacc = tuple(zv for _ in range(_H * nt))
                accs = lax.fori_loop(jlo, jhi, edge_body, zacc)
                for h in range(_H):
                    for t in range(nt):
                        sl = pl.ds(h * _D + t * 16, 16)
                        s_v[sl] = s_v[sl] * f[h] + accs[h * nt + t]
                return (cs + jnp.int32(_CH),
                        newm[0], newm[1], newm[2], newm[3],
                        nd[0], nd[1], nd[2], nd[3])

            fc = lax.fori_loop(0, nch, chunk, carry0)
            for h in range(_H):
                dh = fc[5 + h]
                safe = dh > 0
                invd = 1.0 / jnp.where(safe, dh, 1.0)
                for t in range(nt):
                    r = s_v[pl.ds(h * _D + t * 16, 16)] * invd
                    r = jnp.where(safe, r, 0.0)
                    o = jnp.where(r > 0, r, jnp.exp(r) - 1.0)
                    row_v[pl.ds(h * _D + t * 16, 16)] = o
            pltpu.sync_copy(
                row_v,
                out_h.at[pl.ds(pl.multiple_of((n0 + i) * _H * _D, 8), _H * _D)])
            return tok

        lax.fori_loop(0, _NPW, node_body, 0)

    return sck(emb.reshape(_E * _D), a2t.reshape(_HP * _E)[:_H * _E],
               a1t.reshape(_HP * _NPAD)[:_H * _NPAD], dst)


def kernel(nodes, metapath_instances, metapath_embedding, features, W1, attn2):
    dst = metapath_instances
    emb = metapath_embedding
    w1p = jnp.pad(W1, ((0, _HP - _H), (0, 0)))
    attn2p = jnp.pad(attn2, ((0, _HP - _H), (0, 0)))
    featp = jnp.pad(features, ((0, _NPAD - _N), (0, 0)))
    a2t = _attn_matmul_tc(emb, attn2p)
    a1t = _attn_matmul_tc(featp, w1p)
    out = _sc_aggregate(emb, a2t, a1t, dst)
    return out.reshape(_NPAD, _H * _D)[:_N]


# speedup vs baseline: 9.7539x; 1.1706x over previous
import functools

import jax
import jax.numpy as jnp
from jax import lax
from jax.experimental import pallas as pl
from jax.experimental.pallas import tpu as pltpu
from jax.experimental.pallas import tpu_sc as plsc

_N = 10000
_E = 320000
_H = 4
_D = 128
_ALPHA = 0.2

_NPW = 320
_NPAD = _NPW * 32
_CH = 64
_RPLEN = (_NPAD - _NPW) + _NPW + 16
_HP = 8


def _attn_matmul_tc(x, w):
    m = x.shape[0]
    be = 2560
    assert m % be == 0

    def body(w_ref, x_ref, o_ref):
        o_ref[...] = lax.dot_general(
            w_ref[...], x_ref[...], (((1,), (1,)), ((), ())),
            preferred_element_type=jnp.float32)

    return pl.pallas_call(
        body,
        grid=(m // be,),
        in_specs=[
            pl.BlockSpec((_HP, _D), lambda i: (0, 0)),
            pl.BlockSpec((be, _D), lambda i: (i, 0)),
        ],
        out_specs=pl.BlockSpec((_HP, be), lambda i: (0, i)),
        out_shape=jax.ShapeDtypeStruct((_HP, m), jnp.float32),
    )(w, x)


def _sc_aggregate(emb, a2t, a1t, dst):
    mesh = plsc.VectorSubcoreMesh(core_axis_name="c", subcore_axis_name="s")
    nt = _D // 16
    npw16 = _NPW + 16
    chp = _CH + 16
    chd = 256

    @functools.partial(
        pl.kernel,
        mesh=mesh,
        compiler_params=pltpu.CompilerParams(needs_layout_passes=False),
        out_type=jax.ShapeDtypeStruct((_NPAD * _H * _D,), jnp.float32),
        scratch_types=[
            pltpu.VMEM((npw16,), jnp.int32),
            pltpu.VMEM((_H * npw16,), jnp.float32),
            pltpu.VMEM((_CH * _D,), jnp.float32),
            pltpu.VMEM((_H * _CH,), jnp.float32),
            pltpu.VMEM((_H * chp,), jnp.float32),
            pltpu.VMEM((_H * _D,), jnp.float32),
            pltpu.VMEM((_H * _D,), jnp.float32),
            pltpu.VMEM((16,), jnp.int32),
            pltpu.VMEM((16,), jnp.int32),
            pltpu.VMEM((chd,), jnp.int32),
            pltpu.VMEM((_CH * _D,), jnp.float32),
            pltpu.VMEM((_H * _CH,), jnp.float32),
            pltpu.SemaphoreType.DMA,
            pltpu.SemaphoreType.DMA,
            pltpu.SemaphoreType.DMA,
        ],
    )
    def sck(emb_h, a2_h, a1_h, dst_h, out_h,
            rp_v, a1_v, emb_v, a2_v, w_v, s_v, row_v, p1_v, p2_v, dc_v,
            emb2_v, a22_v, sem, semA, semB):
        wid = lax.axis_index("s") * 2 + lax.axis_index("c")
        n0 = pl.multiple_of(wid * _NPW, 8)
        for h in range(_H):
            pltpu.sync_copy(a1_h.at[pl.ds(pl.multiple_of(h * _NPAD + n0, 8), _NPW)],
                            a1_v.at[pl.ds(h * npw16, _NPW)])
        iot = lax.iota(jnp.int32, 16)
        neg = jnp.float32(-1e30)
        zv = jnp.zeros((16,), jnp.float32)
        perms = [iot ^ s for s in (1, 2, 4, 8)]

        gdn = lax.GatherDimensionNumbers(
            offset_dims=(), collapsed_slice_dims=(0,), start_index_map=(0,))

        def lperm(x, p):
            return lax.gather(
                x, p[:, None], gdn, (1,),
                mode=lax.GatherScatterMode.PROMISE_IN_BOUNDS)

        def allmax(x):
            for p in perms:
                x = jnp.maximum(x, lperm(x, p))
            return x

        def allsum(x):
            for p in perms:
                x = x + lperm(x, p)
            return x

        v1 = n0
        v2 = n0 + jnp.int32(_NPW)
        lane15 = jnp.full((16,), 15, jnp.int32)

        def bs_body(_, c):
            lo1, hi1, lo2, hi2 = c
            mid1 = (lo1 + hi1) // 2
            mid2 = (lo2 + hi2) // 2
            b1 = pl.multiple_of(
                jnp.minimum(mid1 & jnp.int32(-8), jnp.int32(_E - 16)), 8)
            b2 = pl.multiple_of(
                jnp.minimum(mid2 & jnp.int32(-8), jnp.int32(_E - 16)), 8)
            c1 = pltpu.async_copy(dst_h.at[pl.ds(b1, 16)], p1_v, sem)
            c2 = pltpu.async_copy(dst_h.at[pl.ds(b2, 16)], p2_v, sem)
            c1.wait()
            c2.wait()
            val1 = lperm(p1_v[...], jnp.full((16,), mid1 - b1))[0]
            val2 = lperm(p2_v[...], jnp.full((16,), mid2 - b2))[0]
            pr1 = val1 < v1
            pr2 = val2 < v2
            return (jnp.where(pr1, mid1 + 1, lo1), jnp.where(pr1, hi1, mid1),
                    jnp.where(pr2, mid2 + 1, lo2), jnp.where(pr2, hi2, mid2))

        ze = jnp.int32(0)
        e0w, _, eendw, _ = lax.fori_loop(
            0, 19, bs_body, (ze, jnp.int32(_E), ze, jnp.int32(_E)))

        inf = jnp.full((16,), jnp.int32(2147483647))
        for g in range(npw16 // 16):
            rp_v[pl.ds(g * 16, 16)] = inf
        idxm1 = (iot + 15) & 15
        ndch = jnp.where(
            eendw > e0w,
            (eendw - jnp.minimum(e0w & jnp.int32(-8), jnp.int32(_E - chd))
             + jnp.int32(chd - 1)) // chd,
            ze)

        def rp_chunk(_, c):
            pos, prev = c
            cb = pl.multiple_of(
                jnp.minimum(pos & jnp.int32(-8), jnp.int32(_E - chd)), 8)
            pltpu.sync_copy(dst_h.at[pl.ds(cb, chd)], dc_v)
            for g in range(chd // 16):
                v = dc_v[pl.ds(g * 16, 16)]
                shifted = jnp.where(iot == 0, prev, lperm(v, idxm1))
                eidx = cb + g * 16 + iot
                fm = (v != shifted) & (eidx >= pos) & (eidx < eendw)
                sidx = jnp.where(fm, v - n0, jnp.int32(npw16 - 8))
                sval = jnp.where(fm, eidx, jnp.int32(2147483647))
                plsc.store_scatter(rp_v, [sidx], sval)
                prev = lperm(v, lane15)
            return (cb + jnp.int32(chd), prev)

        lax.fori_loop(0, ndch, rp_chunk,
                      (e0w, jnp.full((16,), jnp.int32(-1))))

        carry_f = jnp.full((16,), eendw)
        for k in range(npw16 // 16):
            g = npw16 // 16 - 1 - k
            sl = pl.ds(g * 16, 16)
            y = lax.rev(rp_v[sl], (0,))
            z = -plsc.cummax(-y)
            z = jnp.minimum(z, carry_f)
            carry_f = lperm(z, lane15)
            rp_v[sl] = lax.rev(z, (0,))

        mneg = jnp.full((16,), neg)
        for h in range(_H):
            for t in range(nt):
                s_v[pl.ds(h * _D + t * 16, 16)] = zv
        cs0m = jnp.minimum(e0w & jnp.int32(-8), jnp.int32(_E - _CH))
        nchunks = jnp.where(eendw > e0w,
                            (eendw - cs0m + jnp.int32(_CH - 1)) // _CH, ze)
        nch2 = (nchunks + 1) // 2

        def chunk_cs(k):
            csu = cs0m + k * jnp.int32(_CH)
            cs = pl.multiple_of(
                jnp.minimum(csu, jnp.int32(_E - _CH)), 8)
            return csu, cs

        def issue(k, embbuf, a2buf, sem2):
            _, cs = chunk_cs(k)
            pltpu.async_copy(
                emb_h.at[pl.ds(pl.multiple_of(cs * _D, 8), _CH * _D)],
                embbuf, sem2)
            for hh in range(_H):
                pltpu.async_copy(
                    a2_h.at[pl.ds(pl.multiple_of(hh * _E + cs, 8), _CH)],
                    a2buf.at[pl.ds(hh * _CH, _CH)], sem2)

        def drain(embbuf, a2buf, sem2):
            pltpu.make_async_copy(
                emb_h.at[pl.ds(0, _CH * _D)], embbuf, sem2).wait()
            for hh in range(_H):
                pltpu.make_async_copy(
                    a2_h.at[pl.ds(0, _CH)],
                    a2buf.at[pl.ds(hh * _CH, _CH)], sem2).wait()

        def proc(k, c, embbuf, a2buf):
            i_in = c[0]
            csu, cs = chunk_cs(k)
            pos = jnp.maximum(e0w, csu)
            pos_end = jnp.minimum(csu + jnp.int32(_CH), eendw)

            def bs2(_, lh):
                lo, hi = lh
                mid = (lo + hi) // 2
                val = rp_v[pl.ds(mid, 16)][0]
                pred = val < pos_end
                return (jnp.where(pred, mid + 1, lo),
                        jnp.where(pred, hi, mid))

            lo_f, _ = lax.fori_loop(0, 9, bs2, (ze, jnp.int32(_NPW + 1)))
            t_cnt = lo_f - 1 - i_in + 1

            def nbody(_, cc):
                i = cc[0]
                m = list(cc[1:5])
                d = list(cc[5:9])
                rpv = rp_v[pl.ds(i, 16)]
                e_sn = rpv[0]
                e_en = rpv[1]
                jlo = jnp.maximum(e_sn, pos) - cs
                jhi = jnp.minimum(e_en, pos_end) - cs
                a1b = [jnp.full((16,), a1_v[pl.ds(h * npw16 + i, 16)][0])
                       for h in range(_H)]
                mc = [mneg for _ in range(_H)]
                for h in range(_H):
                    for kk in range(_CH // 16):
                        lidx = kk * 16 + iot
                        valid = (lidx >= jlo) & (lidx < jhi)
                        a = a2buf[pl.ds(h * _CH + kk * 16, 16)] + a1b[h]
                        a = jnp.where(a >= 0, a, _ALPHA * a)
                        a = jnp.where(valid, a, neg)
                        w_v[pl.ds(h * chp + kk * 16, 16)] = a
                        mc[h] = jnp.maximum(mc[h], a)
                newm = []
                f = []
                nd = []
                for h in range(_H):
                    nm = jnp.maximum(m[h], allmax(mc[h]))
                    newm.append(nm)
                    f.append(jnp.exp(m[h] - nm))
                for h in range(_H):
                    acc = zv
                    for kk in range(_CH // 16):
                        sl = pl.ds(h * chp + kk * 16, 16)
                        av = w_v[sl]
                        w = jnp.where(av > neg, jnp.exp(av - newm[h]), zv)
                        w_v[sl] = w
                        acc = acc + w
                    nd.append(d[h] * f[h] + allsum(acc))

                def edge_body(j, accs):
                    accs = list(accs)
                    wj = [jnp.full((16,), w_v[pl.ds(h * chp + j, 16)][0])
                          for h in range(_H)]
                    for t in range(nt):
                        em = embbuf[pl.ds(j * _D + t * 16, 16)]
                        for h in range(_H):
                            accs[h * nt + t] = accs[h * nt + t] + wj[h] * em
                    return tuple(accs)

                zacc = tuple(zv for _ in range(_H * nt))
                accs = lax.fori_loop(jlo, jhi, edge_body, zacc)
                for h in range(_H):
                    for t in range(nt):
                        sl = pl.ds(h * _D + t * 16, 16)
                        s_v[sl] = s_v[sl] * f[h] + accs[h * nt + t]
                ends = e_en <= pos_end

                @pl.when(ends)
                def _finalize():
                    for h in range(_H):
                        dh = nd[h]
                        safe = dh > 0
                        invd = 1.0 / jnp.where(safe, dh, 1.0)
                        for t in range(nt):
                            sl = pl.ds(h * _D + t * 16, 16)
                            r = s_v[sl] * invd
                            r = jnp.where(safe, r, 0.0)
                            o = jnp.where(r > 0, r, jnp.exp(r) - 1.0)
                            row_v[pl.ds(h * _D + t * 16, 16)] = o
                            s_v[sl] = zv
                    pltpu.sync_copy(
                        row_v,
                        out_h.at[pl.ds(
                            pl.multiple_of((n0 + i) * _H * _D, 8), _H * _D)])

                i_n = jnp.where(ends, i + 1, i)
                m_n = [jnp.where(ends, mneg, newm[h]) for h in range(_H)]
                d_n = [jnp.where(ends, zv, nd[h]) for h in range(_H)]
                return (i_n, m_n[0], m_n[1], m_n[2], m_n[3],
                        d_n[0], d_n[1], d_n[2], d_n[3])

            return lax.fori_loop(0, t_cnt, nbody, c)

        carry0 = (ze, mneg, mneg, mneg, mneg, zv, zv, zv, zv)

        def outer(k, c):
            issue(k, emb_v, a2_v, semA)
            drain(emb_v, a2_v, semA)
            return proc(k, c, emb_v, a2_v)

        fcm = lax.fori_loop(0, nchunks, outer, carry0)

        for t in range(nt * _H):
            row_v[pl.ds(t * 16, 16)] = zv

        def zfill(i, tok):
            pltpu.sync_copy(
                row_v,
                out_h.at[pl.ds(
                    pl.multiple_of((n0 + i) * _H * _D, 8), _H * _D)])
            return tok

        lax.fori_loop(fcm[0], jnp.int32(_NPW), zfill, 0)

    return sck(emb.reshape(_E * _D), a2t.reshape(_HP * _E)[:_H * _E],
               a1t.reshape(_HP * _NPAD)[:_H * _NPAD], dst)


def kernel(nodes, metapath_instances, metapath_embedding, features, W1, attn2):
    dst = metapath_instances
    emb = metapath_embedding
    w1p = jnp.pad(W1, ((0, _HP - _H), (0, 0)))
    attn2p = jnp.pad(attn2, ((0, _HP - _H), (0, 0)))
    featp = jnp.pad(features, ((0, _NPAD - _N), (0, 0)))
    a2t = _attn_matmul_tc(emb, attn2p)
    a1t = _attn_matmul_tc(featp, w1p)
    out = _sc_aggregate(emb, a2t, a1t, dst)
    return out.reshape(_NPAD, _H * _D)[:_N]
